# baseline (speedup 1.0000x reference)
import functools

import jax
import jax.numpy as jnp
from jax import lax
from jax.experimental import pallas as pl
from jax.experimental.pallas import tpu as pltpu
from jax.experimental.pallas import tpu_sc as plsc

EMBED = 64
NC = 2
NS = 16
NW = NC * NS
CH = 128
NBUF = 4


@functools.partial(jax.jit, static_argnames=("nch",))
def _sc_gather(seq3, table, nch):
    bpw = nch * CH
    mesh = plsc.VectorSubcoreMesh(core_axis_name="c", subcore_axis_name="s")

    @functools.partial(
        pl.kernel,
        mesh=mesh,
        out_type=jax.ShapeDtypeStruct((NW * bpw, EMBED), jnp.float32),
        scratch_types=[
            pltpu.VMEM((nch, CH), jnp.int32),
            pltpu.VMEM((NBUF, CH, EMBED), jnp.float32),
            pltpu.SemaphoreType.DMA,
        ],
        compiler_params=pltpu.CompilerParams(use_tc_tiling_on_sc=False),
    )
    def k(seq_hbm, tab_hbm, out_hbm, idx_v, rows_v, gsem):
        wid = lax.axis_index("s") * NC + lax.axis_index("c")
        base = wid * bpw
        pltpu.sync_copy(seq_hbm.at[wid], idx_v)

        for b in range(NBUF):
            pltpu.async_copy(tab_hbm.at[idx_v.at[b]], rows_v.at[b], gsem)

        def group(g, carry):
            for b in range(NBUF):
                i = g * NBUF + b
                pltpu.make_async_copy(
                    tab_hbm.at[idx_v.at[i]], rows_v.at[b], gsem
                ).wait()
                pltpu.sync_copy(
                    rows_v.at[b], out_hbm.at[pl.ds(base + i * CH, CH)]
                )
                pltpu.async_copy(
                    tab_hbm.at[idx_v.at[i + NBUF]], rows_v.at[b], gsem
                )
            return carry

        lax.fori_loop(0, nch // NBUF - 1, group, 0)

        for b in range(NBUF):
            i = nch - NBUF + b
            pltpu.make_async_copy(
                tab_hbm.at[idx_v.at[i]], rows_v.at[b], gsem
            ).wait()
            pltpu.sync_copy(rows_v.at[b], out_hbm.at[pl.ds(base + i * CH, CH)])

    return k(seq3, table)


def kernel(sequence, table):
    n_tok = sequence.size
    assert n_tok % (NW * CH * NBUF) == 0
    nch = n_tok // (NW * CH)
    seq3 = sequence.reshape(NW, nch, CH).astype(jnp.int32)
    out = _sc_gather(seq3, table.astype(jnp.float32), nch)
    return out.reshape(sequence.shape + (EMBED,))
